# baseline (speedup 1.0000x reference)
import numpy as np
import jax
import jax.numpy as jnp
from jax import lax
from jax.experimental import pallas as pl
from jax.experimental.pallas import tpu as pltpu

K_PART = 10
M_PART = 16
AJR = 32
NUM_HEAD = 4
HID = 64
N = K_PART * M_PART
DEG = N - 1
HD2 = HID + 2 + K_PART
D_MODEL = NUM_HEAD * HD2

_NEG = np.float32(-3.4e38)
_HI = jax.lax.Precision.HIGHEST


def _dotT(v, W):
    return lax.dot_general(v, W, (((1,), (1,)), ((), ())),
                           preferred_element_type=jnp.float32)


def _prep_kernel(x_ref, label_ref, h0_ref, src_ref, w_ref, et0_ref, et1_ref,
                 d_ref, pe_ref,
                 l0w, l0b, l1w, l1b, l2w, l2b, l3w, l3b, l4w, l4b, l5w, l5b,
                 m0w, m0b, m1w, m1b, m2w, m2b,
                 gs_ref,
                 h_out, hfull_out, a_out, b_out, kk_out, vvl_out):
    f32 = jnp.float32
    x = x_ref[...]
    label = label_ref[...]
    src = src_ref[...]
    wv = w_ref[...]
    et0 = et0_ref[...]
    et1 = et1_ref[...]
    dv = d_ref[...]

    m1 = wv * et0
    m2 = wv * et1
    s1 = jnp.sum(m1, axis=1, keepdims=True)
    s2 = jnp.sum(m2, axis=1, keepdims=True)

    iota_k = lax.broadcasted_iota(jnp.int32, (N, 1, N), 2)
    P1 = jnp.zeros((N, N), f32)
    P2 = jnp.zeros((N, N), f32)
    CH = 8
    for c in range(0, DEG, CH):
        e = min(c + CH, DEG)
        blk = src[:, c:e]
        mask = (blk[:, :, None] == iota_k).astype(f32)
        P1 = P1 + jnp.sum(mask * m1[:, c:e, None], axis=1)
        P2 = P2 + jnp.sum(mask * m2[:, c:e, None], axis=1)

    def topk(v, k):
        out = jnp.zeros((N, k), f32)
        kio = lax.broadcasted_iota(jnp.int32, (1, k), 1)
        jio = lax.broadcasted_iota(jnp.int32, (N, DEG), 1)

        def step(t, carry):
            v, out = carry
            m = jnp.max(v, axis=1, keepdims=True)
            idx = jnp.argmax(v, axis=1)[:, None]
            v = jnp.where(jio == idx, _NEG, v)
            out = out + m * (kio == t).astype(f32)
            return v, out

        _, out = lax.fori_loop(0, k, step, (v, out))
        return out

    n1e = topk(dv * et0, AJR)
    n2e = topk(dv * et1, M_PART - 1)

    base = (_dotT(x, l0w[...]) + l0b[...] + _dotT(label, l1w[...]) + l1b[...]
            + _dotT(n1e, l4w[...]) + l4b[...] + _dotT(n2e, l5w[...]) + l5b[...])

    def gnn_body(_, h):
        n1v = jnp.dot(P1, h, preferred_element_type=f32, precision=_HI) / s1
        n2v = jnp.dot(P2, h, preferred_element_type=f32, precision=_HI) / s2
        return jnp.maximum(
            base + _dotT(n1v, l2w[...]) + l2b[...]
            + _dotT(n2v, l3w[...]) + l3b[...], 0.0)

    h = lax.fori_loop(0, gs_ref[0], gnn_body, h0_ref[...])
    h_out[...] = h

    hfull = jnp.concatenate([h + pe_ref[...], x, label], axis=1)
    hfull_out[...] = hfull

    G = lax.dot_general(hfull, label, (((0,), (0,)), ((), ())),
                        preferred_element_type=f32) / M_PART
    lgc = lax.dot_general(label, G, (((1,), (1,)), ((), ())),
                          preferred_element_type=f32)

    W0 = m0w[...]
    a_out[...] = _dotT(hfull, W0[:, :HD2]) + _dotT(lgc, W0[:, 2 * HD2:3 * HD2])
    b_out[...] = _dotT(hfull, W0[:, HD2:2 * HD2]) + _dotT(lgc, W0[:, 3 * HD2:])

    W1 = m1w[...]
    kk_out[...] = (_dotT(hfull, W1[:, :HD2]) + _dotT(hfull, W1[:, HD2:2 * HD2])
                   + _dotT(hfull, W1[:, 2 * HD2:3 * HD2])
                   + _dotT(hfull, W1[:, 3 * HD2:]) + m1b[...])
    W2 = m2w[...]
    vvl_out[...] = (_dotT(hfull, W2[:, :HD2]) + _dotT(hfull, W2[:, HD2:2 * HD2])
                    + _dotT(hfull, W2[:, 2 * HD2:3 * HD2])
                    + _dotT(hfull, W2[:, 3 * HD2:]) + m2b[...])


_BI = 8


def _attn_kernel(a_ref, b_ref, b0_ref, kk_ref, vvl_ref, m3w, m3b,
                 v1w, v1b, v2w, v2b, s_out, q_out):
    f32 = jnp.float32
    bf16 = jnp.bfloat16
    i0 = pl.program_id(0) * _BI
    A = a_ref[...]
    B = b_ref[...]
    b0 = b0_ref[...]
    kk = kk_ref[...]
    vvl = vvl_ref[...]
    isq = np.float32(np.sqrt(HD2))
    v2r = v2w[...].astype(bf16).astype(f32)

    def soft(s):
        m = jnp.max(s, axis=-1, keepdims=True)
        p = jnp.exp(s - m)
        return p / jnp.sum(p, axis=-1, keepdims=True)

    for li in range(_BI):
        a_i = a_ref[pl.ds(i0 + li, 1), :] + b0
        bq_i = b_ref[pl.ds(i0 + li, 1), :] + b0
        qq1 = a_i + B
        qq2 = bq_i + A
        xo1_parts, xo2_parts = [], []
        for hh in range(NUM_HEAD):
            sl = slice(hh * HD2, (hh + 1) * HD2)
            kk_h = kk[:, sl]
            vv_h = vvl[:, sl]
            p1 = soft(_dotT(qq1[:, sl], kk_h) / isq)
            p2 = soft(_dotT(qq2[:, sl], kk_h) / isq)
            xo1_parts.append(jnp.dot(p1, vv_h, preferred_element_type=f32))
            xo2_parts.append(jnp.dot(p2, vv_h, preferred_element_type=f32))
        xo1 = jnp.concatenate(xo1_parts, axis=1)
        xo2 = jnp.concatenate(xo2_parts, axis=1)
        S = (_dotT(xo1, m3w[...]) + m3b[...]) + (_dotT(xo2, m3w[...]) + m3b[...])
        s_out[li * N:(li + 1) * N, 0, :] = S
        r = jnp.maximum(_dotT(S, v1w[...]) + v1b[...], 0.0)
        r = r.astype(bf16).astype(f32)
        q = jnp.sum(r * v2r, axis=1, keepdims=True) + v2b[0, 0]
        q_out[li * N:(li + 1) * N, :] = q


def _pe_table():
    dm = HID
    pos = np.arange(50)[:, None].astype(np.float32)
    div = np.exp(np.arange(0, dm, 2).astype(np.float32) * -(np.log(10000.0) / dm))
    pe = np.zeros((50, dm), dtype=np.float32)
    pe[:, 0::2] = np.sin(pos * div)
    pe[:, 1::2] = np.cos(pos * div)
    return pe


_PE = _pe_table()


def kernel(x, label, h0, src, w, e_type, d,
           l0_W, l0_b, l1_W, l1_b, l2_W, l2_b, l3_W, l3_b, l4_W, l4_b,
           l5_W, l5_b,
           mha0_W, mha0_b, mha1_W, mha1_b, mha2_W, mha2_b, mha3_W, mha3_b,
           v1_W, v1_b, v2_W, v2_b, gnn_step, max_step, remain_step):
    f32 = jnp.float32
    src = src.astype(jnp.int32)
    w2 = w[:, :, 0]
    et0 = e_type[:, :, 0]
    et1 = e_type[:, :, 1]
    d2 = d[:, :, 0]
    pe_row = jnp.asarray(_PE)[remain_step + 0 * max_step][None, :]
    gs = jnp.asarray(gnn_step, jnp.int32).reshape(1)

    row = lambda b: jnp.asarray(b, f32).reshape(1, -1)

    vmem = pl.BlockSpec(memory_space=pltpu.VMEM)
    n_in = 28
    h, hfull, A, B, kk, vvl = pl.pallas_call(
        _prep_kernel,
        out_shape=(
            jax.ShapeDtypeStruct((N, HID), f32),
            jax.ShapeDtypeStruct((N, HD2), f32),
            jax.ShapeDtypeStruct((N, D_MODEL), f32),
            jax.ShapeDtypeStruct((N, D_MODEL), f32),
            jax.ShapeDtypeStruct((N, D_MODEL), f32),
            jax.ShapeDtypeStruct((N, D_MODEL), f32),
        ),
        in_specs=[vmem] * (n_in - 1) + [pl.BlockSpec(memory_space=pltpu.SMEM)],
        out_specs=(vmem,) * 6,
    )(x, label, h0, src, w2, et0, et1, d2, pe_row,
      l0_W, row(l0_b), l1_W, row(l1_b), l2_W, row(l2_b), l3_W, row(l3_b),
      l4_W, row(l4_b), l5_W, row(l5_b),
      mha0_W, row(mha0_b), mha1_W, row(mha1_b), mha2_W, row(mha2_b), gs)

    grid = N // _BI
    full = lambda shape: pl.BlockSpec(shape, lambda i: (0,) * len(shape))
    S2, Qs = pl.pallas_call(
        _attn_kernel,
        grid=(grid,),
        in_specs=[
            full((N, D_MODEL)),
            full((N, D_MODEL)),
            full((1, D_MODEL)),
            full((N, D_MODEL)),
            full((N, D_MODEL)),
            full((D_MODEL, D_MODEL)),
            full((1, D_MODEL)),
            full((HID // 2, D_MODEL)),
            full((1, HID // 2)),
            full((1, HID // 2)),
            full((1, 1)),
        ],
        out_specs=(
            pl.BlockSpec((_BI * N, 1, D_MODEL), lambda i: (i, 0, 0)),
            pl.BlockSpec((_BI * N, 1), lambda i: (i, 0)),
        ),
        out_shape=(
            jax.ShapeDtypeStruct((N * N, 1, D_MODEL), f32),
            jax.ShapeDtypeStruct((N * N, 1), f32),
        ),
    )(A, B, row(mha0_b), kk, vvl, mha3_W, row(mha3_b), v1_W, row(v1_b),
      row(v2_W), jnp.asarray(v2_b, f32).reshape(1, 1))

    return (S2, h, hfull, Qs.reshape(N * N))


# speedup vs baseline: 1.4904x; 1.4904x over previous
import numpy as np
import jax
import jax.numpy as jnp
from jax import lax
from jax.experimental import pallas as pl
from jax.experimental.pallas import tpu as pltpu

K_PART = 10
M_PART = 16
AJR = 32
NUM_HEAD = 4
HID = 64
N = K_PART * M_PART
DEG = N - 1
HD2 = HID + 2 + K_PART
D_MODEL = NUM_HEAD * HD2

_NEG = np.float32(-3.4e38)
_HI = jax.lax.Precision.HIGHEST


def _dotT(v, W):
    return lax.dot_general(v, W, (((1,), (1,)), ((), ())),
                           preferred_element_type=jnp.float32)


def _prep_kernel(x_ref, label_ref, h0_ref, src_ref, w_ref, et0_ref, et1_ref,
                 d_ref, pe_ref,
                 l0w, l0b, l1w, l1b, l2w, l2b, l3w, l3b, l4w, l4b, l5w, l5b,
                 m0w, m0b, m1w, m1b, m2w, m2b,
                 gs_ref,
                 h_out, hfull_out, a_out, b_out, kk_out, vvl_out):
    f32 = jnp.float32
    x = x_ref[...]
    label = label_ref[...]
    src = src_ref[...]
    wv = w_ref[...]
    et0 = et0_ref[...]
    et1 = et1_ref[...]
    dv = d_ref[...]

    m1 = wv * et0
    m2 = wv * et1
    s1 = jnp.sum(m1, axis=1, keepdims=True)
    s2 = jnp.sum(m2, axis=1, keepdims=True)

    iota_k = lax.broadcasted_iota(jnp.int32, (N, 1, N), 2)
    P1 = jnp.zeros((N, N), f32)
    P2 = jnp.zeros((N, N), f32)
    CH = 8
    for c in range(0, DEG, CH):
        e = min(c + CH, DEG)
        blk = src[:, c:e]
        mask = (blk[:, :, None] == iota_k).astype(f32)
        P1 = P1 + jnp.sum(mask * m1[:, c:e, None], axis=1)
        P2 = P2 + jnp.sum(mask * m2[:, c:e, None], axis=1)

    def topk(v, k):
        out = jnp.zeros((N, k), f32)
        kio = lax.broadcasted_iota(jnp.int32, (1, k), 1)
        jio = lax.broadcasted_iota(jnp.int32, (N, DEG), 1)

        def step(t, carry):
            v, out = carry
            m = jnp.max(v, axis=1, keepdims=True)
            idx = jnp.argmax(v, axis=1)[:, None]
            v = jnp.where(jio == idx, _NEG, v)
            out = out + m * (kio == t).astype(f32)
            return v, out

        _, out = lax.fori_loop(0, k, step, (v, out))
        return out

    n1e = topk(dv * et0, AJR)
    n2e = topk(dv * et1, M_PART - 1)

    base = (_dotT(x, l0w[...]) + l0b[...] + _dotT(label, l1w[...]) + l1b[...]
            + _dotT(n1e, l4w[...]) + l4b[...] + _dotT(n2e, l5w[...]) + l5b[...])

    def gnn_body(_, h):
        n1v = jnp.dot(P1, h, preferred_element_type=f32, precision=_HI) / s1
        n2v = jnp.dot(P2, h, preferred_element_type=f32, precision=_HI) / s2
        return jnp.maximum(
            base + _dotT(n1v, l2w[...]) + l2b[...]
            + _dotT(n2v, l3w[...]) + l3b[...], 0.0)

    h = lax.fori_loop(0, gs_ref[0], gnn_body, h0_ref[...])
    h_out[...] = h

    hfull = jnp.concatenate([h + pe_ref[...], x, label], axis=1)
    hfull_out[...] = hfull

    G = lax.dot_general(hfull, label, (((0,), (0,)), ((), ())),
                        preferred_element_type=f32) / M_PART
    lgc = lax.dot_general(label, G, (((1,), (1,)), ((), ())),
                          preferred_element_type=f32)

    W0 = m0w[...]
    a_out[...] = _dotT(hfull, W0[:, :HD2]) + _dotT(lgc, W0[:, 2 * HD2:3 * HD2])
    b_out[...] = _dotT(hfull, W0[:, HD2:2 * HD2]) + _dotT(lgc, W0[:, 3 * HD2:])

    W1 = m1w[...]
    kk_out[...] = (_dotT(hfull, W1[:, :HD2]) + _dotT(hfull, W1[:, HD2:2 * HD2])
                   + _dotT(hfull, W1[:, 2 * HD2:3 * HD2])
                   + _dotT(hfull, W1[:, 3 * HD2:]) + m1b[...])
    W2 = m2w[...]
    vvl_out[...] = (_dotT(hfull, W2[:, :HD2]) + _dotT(hfull, W2[:, HD2:2 * HD2])
                    + _dotT(hfull, W2[:, 2 * HD2:3 * HD2])
                    + _dotT(hfull, W2[:, 3 * HD2:]) + m2b[...])


_BI = 8


def _attn_kernel(a_ref, b_ref, b0_ref, kk_ref, vvl_ref, m3w, m3b,
                 v1w, v1b, v2w, v2b, s_out, q_out):
    f32 = jnp.float32
    bf16 = jnp.bfloat16
    i0 = pl.program_id(0) * _BI
    A = a_ref[...]
    B = b_ref[...]
    b0 = b0_ref[...]
    kk = kk_ref[...]
    vvl = vvl_ref[...]
    isq = np.float32(np.sqrt(HD2))
    v2r = v2w[...].astype(bf16).astype(f32)

    def soft(s):
        m = jnp.max(s, axis=-1, keepdims=True)
        p = jnp.exp(s - m)
        return p / jnp.sum(p, axis=-1, keepdims=True)

    qq1 = jnp.concatenate(
        [(a_ref[pl.ds(i0 + li, 1), :] + b0) + B for li in range(_BI)], axis=0)
    qq2 = jnp.concatenate(
        [(b_ref[pl.ds(i0 + li, 1), :] + b0) + A for li in range(_BI)], axis=0)
    xo1_parts, xo2_parts = [], []
    for hh in range(NUM_HEAD):
        sl = slice(hh * HD2, (hh + 1) * HD2)
        kk_h = kk[:, sl]
        vv_h = vvl[:, sl]
        p1 = soft(_dotT(qq1[:, sl], kk_h) / isq)
        p2 = soft(_dotT(qq2[:, sl], kk_h) / isq)
        xo1_parts.append(jnp.dot(p1, vv_h, preferred_element_type=f32))
        xo2_parts.append(jnp.dot(p2, vv_h, preferred_element_type=f32))
    xo1 = jnp.concatenate(xo1_parts, axis=1)
    xo2 = jnp.concatenate(xo2_parts, axis=1)
    S = (_dotT(xo1, m3w[...]) + m3b[...]) + (_dotT(xo2, m3w[...]) + m3b[...])
    s_out[:, 0, :] = S
    r = jnp.maximum(_dotT(S, v1w[...]) + v1b[...], 0.0)
    r = r.astype(bf16).astype(f32)
    q = jnp.sum(r * v2r, axis=1, keepdims=True) + v2b[0, 0]
    q_out[...] = q


def _pe_table():
    dm = HID
    pos = np.arange(50)[:, None].astype(np.float32)
    div = np.exp(np.arange(0, dm, 2).astype(np.float32) * -(np.log(10000.0) / dm))
    pe = np.zeros((50, dm), dtype=np.float32)
    pe[:, 0::2] = np.sin(pos * div)
    pe[:, 1::2] = np.cos(pos * div)
    return pe


_PE = _pe_table()


def kernel(x, label, h0, src, w, e_type, d,
           l0_W, l0_b, l1_W, l1_b, l2_W, l2_b, l3_W, l3_b, l4_W, l4_b,
           l5_W, l5_b,
           mha0_W, mha0_b, mha1_W, mha1_b, mha2_W, mha2_b, mha3_W, mha3_b,
           v1_W, v1_b, v2_W, v2_b, gnn_step, max_step, remain_step):
    f32 = jnp.float32
    src = src.astype(jnp.int32)
    w2 = w[:, :, 0]
    et0 = e_type[:, :, 0]
    et1 = e_type[:, :, 1]
    d2 = d[:, :, 0]
    pe_row = jnp.asarray(_PE)[remain_step + 0 * max_step][None, :]
    gs = jnp.asarray(gnn_step, jnp.int32).reshape(1)

    row = lambda b: jnp.asarray(b, f32).reshape(1, -1)

    vmem = pl.BlockSpec(memory_space=pltpu.VMEM)
    n_in = 28
    h, hfull, A, B, kk, vvl = pl.pallas_call(
        _prep_kernel,
        out_shape=(
            jax.ShapeDtypeStruct((N, HID), f32),
            jax.ShapeDtypeStruct((N, HD2), f32),
            jax.ShapeDtypeStruct((N, D_MODEL), f32),
            jax.ShapeDtypeStruct((N, D_MODEL), f32),
            jax.ShapeDtypeStruct((N, D_MODEL), f32),
            jax.ShapeDtypeStruct((N, D_MODEL), f32),
        ),
        in_specs=[vmem] * (n_in - 1) + [pl.BlockSpec(memory_space=pltpu.SMEM)],
        out_specs=(vmem,) * 6,
    )(x, label, h0, src, w2, et0, et1, d2, pe_row,
      l0_W, row(l0_b), l1_W, row(l1_b), l2_W, row(l2_b), l3_W, row(l3_b),
      l4_W, row(l4_b), l5_W, row(l5_b),
      mha0_W, row(mha0_b), mha1_W, row(mha1_b), mha2_W, row(mha2_b), gs)

    grid = N // _BI
    full = lambda shape: pl.BlockSpec(shape, lambda i: (0,) * len(shape))
    S2, Qs = pl.pallas_call(
        _attn_kernel,
        grid=(grid,),
        in_specs=[
            full((N, D_MODEL)),
            full((N, D_MODEL)),
            full((1, D_MODEL)),
            full((N, D_MODEL)),
            full((N, D_MODEL)),
            full((D_MODEL, D_MODEL)),
            full((1, D_MODEL)),
            full((HID // 2, D_MODEL)),
            full((1, HID // 2)),
            full((1, HID // 2)),
            full((1, 1)),
        ],
        out_specs=(
            pl.BlockSpec((_BI * N, 1, D_MODEL), lambda i: (i, 0, 0)),
            pl.BlockSpec((_BI * N, 1), lambda i: (i, 0)),
        ),
        out_shape=(
            jax.ShapeDtypeStruct((N * N, 1, D_MODEL), f32),
            jax.ShapeDtypeStruct((N * N, 1), f32),
        ),
    )(A, B, row(mha0_b), kk, vvl, mha3_W, row(mha3_b), v1_W, row(v1_b),
      row(v2_W), jnp.asarray(v2_b, f32).reshape(1, 1))

    return (S2, h, hfull, Qs.reshape(N * N))


# speedup vs baseline: 1.7511x; 1.1749x over previous
import numpy as np
import jax
import jax.numpy as jnp
from jax import lax
from jax.experimental import pallas as pl
from jax.experimental.pallas import tpu as pltpu

K_PART = 10
M_PART = 16
AJR = 32
NUM_HEAD = 4
HID = 64
N = K_PART * M_PART
DEG = N - 1
HD2 = HID + 2 + K_PART
D_MODEL = NUM_HEAD * HD2

_NEG = np.float32(-3.4e38)
_HI = jax.lax.Precision.HIGHEST


def _dotT(v, W):
    return lax.dot_general(v, W, (((1,), (1,)), ((), ())),
                           preferred_element_type=jnp.float32)


def _prep_kernel(x_ref, label_ref, h0_ref, src_ref, w_ref, et0_ref, et1_ref,
                 d_ref, pe_ref,
                 l0w, l0b, l1w, l1b, l2w, l2b, l3w, l3b, l4w, l4b, l5w, l5b,
                 m0w, m0b, m1w, m1b, m2w, m2b,
                 gs_ref,
                 h_out, hfull_out, a_out, b_out, kk_out, vvl_out):
    f32 = jnp.float32
    x = x_ref[...]
    label = label_ref[...]
    src = src_ref[...]
    wv = w_ref[...]
    et0 = et0_ref[...]
    et1 = et1_ref[...]
    dv = d_ref[...]

    m1 = wv * et0
    m2 = wv * et1
    s1 = jnp.sum(m1, axis=1, keepdims=True)
    s2 = jnp.sum(m2, axis=1, keepdims=True)

    iota_k = lax.broadcasted_iota(jnp.int32, (N, 1, N), 2)
    P1 = jnp.zeros((N, N), f32)
    P2 = jnp.zeros((N, N), f32)
    CH = 8
    for c in range(0, DEG, CH):
        e = min(c + CH, DEG)
        blk = src[:, c:e]
        mask = (blk[:, :, None] == iota_k).astype(f32)
        P1 = P1 + jnp.sum(mask * m1[:, c:e, None], axis=1)
        P2 = P2 + jnp.sum(mask * m2[:, c:e, None], axis=1)

    def topk(v, k):
        out = jnp.zeros((N, k), f32)
        kio = lax.broadcasted_iota(jnp.int32, (1, k), 1)
        jio = lax.broadcasted_iota(jnp.int32, (N, DEG), 1)

        def step(t, carry):
            v, out = carry
            m = jnp.max(v, axis=1, keepdims=True)
            idx = jnp.argmax(v, axis=1)[:, None]
            v = jnp.where(jio == idx, _NEG, v)
            out = out + m * (kio == t).astype(f32)
            return v, out

        _, out = lax.fori_loop(0, k, step, (v, out))
        return out

    n1e = topk(dv * et0, AJR)
    n2e = topk(dv * et1, M_PART - 1)

    base = (_dotT(x, l0w[...]) + l0b[...] + _dotT(label, l1w[...]) + l1b[...]
            + _dotT(n1e, l4w[...]) + l4b[...] + _dotT(n2e, l5w[...]) + l5b[...])

    def gnn_body(_, h):
        n1v = jnp.dot(P1, h, preferred_element_type=f32, precision=_HI) / s1
        n2v = jnp.dot(P2, h, preferred_element_type=f32, precision=_HI) / s2
        return jnp.maximum(
            base + _dotT(n1v, l2w[...]) + l2b[...]
            + _dotT(n2v, l3w[...]) + l3b[...], 0.0)

    h = lax.fori_loop(0, gs_ref[0], gnn_body, h0_ref[...])
    h_out[...] = h

    hfull = jnp.concatenate([h + pe_ref[...], x, label], axis=1)
    hfull_out[...] = hfull

    G = lax.dot_general(hfull, label, (((0,), (0,)), ((), ())),
                        preferred_element_type=f32) / M_PART
    lgc = lax.dot_general(label, G, (((1,), (1,)), ((), ())),
                          preferred_element_type=f32)

    W0 = m0w[...]
    a_out[...] = _dotT(hfull, W0[:, :HD2]) + _dotT(lgc, W0[:, 2 * HD2:3 * HD2])
    b_out[...] = _dotT(hfull, W0[:, HD2:2 * HD2]) + _dotT(lgc, W0[:, 3 * HD2:])

    W1 = m1w[...]
    kk_out[...] = (_dotT(hfull, W1[:, :HD2]) + _dotT(hfull, W1[:, HD2:2 * HD2])
                   + _dotT(hfull, W1[:, 2 * HD2:3 * HD2])
                   + _dotT(hfull, W1[:, 3 * HD2:]) + m1b[...])
    W2 = m2w[...]
    vvl_out[...] = (_dotT(hfull, W2[:, :HD2]) + _dotT(hfull, W2[:, HD2:2 * HD2])
                    + _dotT(hfull, W2[:, 2 * HD2:3 * HD2])
                    + _dotT(hfull, W2[:, 3 * HD2:]) + m2b[...])


_BI = 16


def _attn_kernel(a_ref, b_ref, b0_ref, kk_ref, vvl_ref, m3w, m3b,
                 v1w, v1b, v2w, v2b, s_out, q_out):
    f32 = jnp.float32
    bf16 = jnp.bfloat16
    i0 = pl.program_id(0) * _BI
    A = a_ref[...]
    B = b_ref[...]
    b0 = b0_ref[...]
    kk = kk_ref[...]
    vvl = vvl_ref[...]
    isq = np.float32(np.sqrt(HD2))
    v2r = v2w[...].astype(bf16).astype(f32)

    def soft(s):
        m = jnp.max(s, axis=-1, keepdims=True)
        p = jnp.exp(s - m)
        return p / jnp.sum(p, axis=-1, keepdims=True)

    qq1 = jnp.concatenate(
        [(a_ref[pl.ds(i0 + li, 1), :] + b0) + B for li in range(_BI)], axis=0)
    qq2 = jnp.concatenate(
        [(b_ref[pl.ds(i0 + li, 1), :] + b0) + A for li in range(_BI)], axis=0)
    xo1_parts, xo2_parts = [], []
    for hh in range(NUM_HEAD):
        sl = slice(hh * HD2, (hh + 1) * HD2)
        kk_h = kk[:, sl]
        vv_h = vvl[:, sl]
        p1 = soft(_dotT(qq1[:, sl], kk_h) / isq)
        p2 = soft(_dotT(qq2[:, sl], kk_h) / isq)
        xo1_parts.append(jnp.dot(p1, vv_h, preferred_element_type=f32))
        xo2_parts.append(jnp.dot(p2, vv_h, preferred_element_type=f32))
    xo1 = jnp.concatenate(xo1_parts, axis=1)
    xo2 = jnp.concatenate(xo2_parts, axis=1)
    S = (_dotT(xo1, m3w[...]) + m3b[...]) + (_dotT(xo2, m3w[...]) + m3b[...])
    s_out[...] = S
    r = jnp.maximum(_dotT(S, v1w[...]) + v1b[...], 0.0)
    r = r.astype(bf16).astype(f32)
    q = jnp.sum(r * v2r, axis=1, keepdims=True) + v2b[0, 0]
    q_out[...] = q


def _pe_table():
    dm = HID
    pos = np.arange(50)[:, None].astype(np.float32)
    div = np.exp(np.arange(0, dm, 2).astype(np.float32) * -(np.log(10000.0) / dm))
    pe = np.zeros((50, dm), dtype=np.float32)
    pe[:, 0::2] = np.sin(pos * div)
    pe[:, 1::2] = np.cos(pos * div)
    return pe


_PE = _pe_table()


def kernel(x, label, h0, src, w, e_type, d,
           l0_W, l0_b, l1_W, l1_b, l2_W, l2_b, l3_W, l3_b, l4_W, l4_b,
           l5_W, l5_b,
           mha0_W, mha0_b, mha1_W, mha1_b, mha2_W, mha2_b, mha3_W, mha3_b,
           v1_W, v1_b, v2_W, v2_b, gnn_step, max_step, remain_step):
    f32 = jnp.float32
    src = src.astype(jnp.int32)
    w2 = w[:, :, 0]
    et0 = e_type[:, :, 0]
    et1 = e_type[:, :, 1]
    d2 = d[:, :, 0]
    pe_row = jnp.asarray(_PE)[remain_step + 0 * max_step][None, :]
    gs = jnp.asarray(gnn_step, jnp.int32).reshape(1)

    row = lambda b: jnp.asarray(b, f32).reshape(1, -1)

    vmem = pl.BlockSpec(memory_space=pltpu.VMEM)
    n_in = 28
    h, hfull, A, B, kk, vvl = pl.pallas_call(
        _prep_kernel,
        out_shape=(
            jax.ShapeDtypeStruct((N, HID), f32),
            jax.ShapeDtypeStruct((N, HD2), f32),
            jax.ShapeDtypeStruct((N, D_MODEL), f32),
            jax.ShapeDtypeStruct((N, D_MODEL), f32),
            jax.ShapeDtypeStruct((N, D_MODEL), f32),
            jax.ShapeDtypeStruct((N, D_MODEL), f32),
        ),
        in_specs=[vmem] * (n_in - 1) + [pl.BlockSpec(memory_space=pltpu.SMEM)],
        out_specs=(vmem,) * 6,
    )(x, label, h0, src, w2, et0, et1, d2, pe_row,
      l0_W, row(l0_b), l1_W, row(l1_b), l2_W, row(l2_b), l3_W, row(l3_b),
      l4_W, row(l4_b), l5_W, row(l5_b),
      mha0_W, row(mha0_b), mha1_W, row(mha1_b), mha2_W, row(mha2_b), gs)

    grid = N // _BI
    full = lambda shape: pl.BlockSpec(shape, lambda i: (0,) * len(shape))
    S2, Qs = pl.pallas_call(
        _attn_kernel,
        grid=(grid,),
        in_specs=[
            full((N, D_MODEL)),
            full((N, D_MODEL)),
            full((1, D_MODEL)),
            full((N, D_MODEL)),
            full((N, D_MODEL)),
            full((D_MODEL, D_MODEL)),
            full((1, D_MODEL)),
            full((HID // 2, D_MODEL)),
            full((1, HID // 2)),
            full((1, HID // 2)),
            full((1, 1)),
        ],
        out_specs=(
            pl.BlockSpec((_BI * N, D_MODEL), lambda i: (i, 0)),
            pl.BlockSpec((_BI * N, 1), lambda i: (i, 0)),
        ),
        out_shape=(
            jax.ShapeDtypeStruct((N * N, D_MODEL), f32),
            jax.ShapeDtypeStruct((N * N, 1), f32),
        ),
    )(A, B, row(mha0_b), kk, vvl, mha3_W, row(mha3_b), v1_W, row(v1_b),
      row(v2_W), jnp.asarray(v2_b, f32).reshape(1, 1))

    return (S2.reshape(N * N, 1, D_MODEL), h, hfull, Qs.reshape(N * N))


# speedup vs baseline: 1.9178x; 1.0952x over previous
import numpy as np
import jax
import jax.numpy as jnp
from jax import lax
from jax.experimental import pallas as pl
from jax.experimental.pallas import tpu as pltpu

K_PART = 10
M_PART = 16
AJR = 32
NUM_HEAD = 4
HID = 64
N = K_PART * M_PART
DEG = N - 1
HD2 = HID + 2 + K_PART
D_MODEL = NUM_HEAD * HD2

_NEG = np.float32(-3.4e38)
_HI = jax.lax.Precision.HIGHEST


def _dotT(v, W):
    return lax.dot_general(v, W, (((1,), (1,)), ((), ())),
                           preferred_element_type=jnp.float32)


def _prep_kernel(x_ref, label_ref, h0_ref, src_ref, w_ref, et0_ref, et1_ref,
                 d_ref, pe_ref,
                 l0w, l0b, l1w, l1b, l2w, l2b, l3w, l3b, l4w, l4b, l5w, l5b,
                 m0w, m0b, m1w, m1b, m2w, m2b,
                 gs_ref,
                 h_out, hfull_out, a_out, b_out, kk_out, vvl_out):
    f32 = jnp.float32
    x = x_ref[...]
    label = label_ref[...]
    src = src_ref[...]
    wv = w_ref[...]
    et0 = et0_ref[...]
    et1 = et1_ref[...]
    dv = d_ref[...]

    m1 = wv * et0
    m2 = wv * et1
    s1 = jnp.sum(m1, axis=1, keepdims=True)
    s2 = jnp.sum(m2, axis=1, keepdims=True)

    iota_k = lax.broadcasted_iota(jnp.int32, (N, 1, N), 2)
    P1 = jnp.zeros((N, N), f32)
    P2 = jnp.zeros((N, N), f32)
    CH = 8
    for c in range(0, DEG, CH):
        e = min(c + CH, DEG)
        blk = src[:, c:e]
        mask = (blk[:, :, None] == iota_k).astype(f32)
        P1 = P1 + jnp.sum(mask * m1[:, c:e, None], axis=1)
        P2 = P2 + jnp.sum(mask * m2[:, c:e, None], axis=1)

    def topk(v, k):
        out = jnp.zeros((N, k), f32)
        kio = lax.broadcasted_iota(jnp.int32, (1, k), 1)
        jio = lax.broadcasted_iota(jnp.int32, (N, DEG), 1)

        def step(t, carry):
            v, out = carry
            m = jnp.max(v, axis=1, keepdims=True)
            idx = jnp.argmax(v, axis=1)[:, None]
            v = jnp.where(jio == idx, _NEG, v)
            out = out + m * (kio == t).astype(f32)
            return v, out

        _, out = lax.fori_loop(0, k, step, (v, out))
        return out

    n1e = topk(dv * et0, AJR)
    n2e = topk(dv * et1, M_PART - 1)

    base = (_dotT(x, l0w[...]) + l0b[...] + _dotT(label, l1w[...]) + l1b[...]
            + _dotT(n1e, l4w[...]) + l4b[...] + _dotT(n2e, l5w[...]) + l5b[...])

    def gnn_body(_, h):
        n1v = jnp.dot(P1, h, preferred_element_type=f32, precision=_HI) / s1
        n2v = jnp.dot(P2, h, preferred_element_type=f32, precision=_HI) / s2
        return jnp.maximum(
            base + _dotT(n1v, l2w[...]) + l2b[...]
            + _dotT(n2v, l3w[...]) + l3b[...], 0.0)

    h = lax.fori_loop(0, gs_ref[0], gnn_body, h0_ref[...])
    h_out[...] = h

    hfull = jnp.concatenate([h + pe_ref[...], x, label], axis=1)
    hfull_out[...] = hfull

    G = lax.dot_general(hfull, label, (((0,), (0,)), ((), ())),
                        preferred_element_type=f32) / M_PART
    lgc = lax.dot_general(label, G, (((1,), (1,)), ((), ())),
                          preferred_element_type=f32)

    W0 = m0w[...]
    a_out[...] = _dotT(hfull, W0[:, :HD2]) + _dotT(lgc, W0[:, 2 * HD2:3 * HD2])
    b_out[...] = _dotT(hfull, W0[:, HD2:2 * HD2]) + _dotT(lgc, W0[:, 3 * HD2:])

    W1 = m1w[...]
    kk_out[...] = (_dotT(hfull, W1[:, :HD2]) + _dotT(hfull, W1[:, HD2:2 * HD2])
                   + _dotT(hfull, W1[:, 2 * HD2:3 * HD2])
                   + _dotT(hfull, W1[:, 3 * HD2:]) + m1b[...])
    W2 = m2w[...]
    vvl_out[...] = (_dotT(hfull, W2[:, :HD2]) + _dotT(hfull, W2[:, HD2:2 * HD2])
                    + _dotT(hfull, W2[:, 2 * HD2:3 * HD2])
                    + _dotT(hfull, W2[:, 3 * HD2:]) + m2b[...])


_BI = 16


def _attn_kernel(a_ref, b_ref, b0_ref, kk_ref, vvl_ref, m3w, m3b, m_out):
    f32 = jnp.float32
    i0 = pl.program_id(0) * _BI
    B = b_ref[...]
    b0 = b0_ref[...]
    kk = kk_ref[...]
    vvl = vvl_ref[...]
    isq = np.float32(np.sqrt(HD2))

    def soft(s):
        m = jnp.max(s, axis=-1, keepdims=True)
        p = jnp.exp(s - m)
        return p / jnp.sum(p, axis=-1, keepdims=True)

    qq1 = jnp.concatenate(
        [(a_ref[pl.ds(i0 + li, 1), :] + b0) + B for li in range(_BI)], axis=0)
    xo1_parts = []
    for hh in range(NUM_HEAD):
        sl = slice(hh * HD2, (hh + 1) * HD2)
        p1 = soft(_dotT(qq1[:, sl], kk[:, sl]) / isq)
        xo1_parts.append(jnp.dot(p1, vvl[:, sl], preferred_element_type=f32))
    xo1 = jnp.concatenate(xo1_parts, axis=1)
    m_out[...] = _dotT(xo1, m3w[...]) + m3b[...]


def _combine_kernel(mr_ref, mc_ref, v1w, v1b, v2w, v2b, s_out, q_out):
    f32 = jnp.float32
    bf16 = jnp.bfloat16
    mc = mc_ref[...]
    mt = jnp.concatenate([mc[:, li, :] for li in range(_BI)], axis=0)
    S = mr_ref[...] + mt
    s_out[...] = S
    r = jnp.maximum(_dotT(S, v1w[...]) + v1b[...], 0.0)
    r = r.astype(bf16).astype(f32)
    v2r = v2w[...].astype(bf16).astype(f32)
    q = jnp.sum(r * v2r, axis=1, keepdims=True) + v2b[0, 0]
    q_out[...] = q


def _pe_table():
    dm = HID
    pos = np.arange(50)[:, None].astype(np.float32)
    div = np.exp(np.arange(0, dm, 2).astype(np.float32) * -(np.log(10000.0) / dm))
    pe = np.zeros((50, dm), dtype=np.float32)
    pe[:, 0::2] = np.sin(pos * div)
    pe[:, 1::2] = np.cos(pos * div)
    return pe


_PE = _pe_table()


def kernel(x, label, h0, src, w, e_type, d,
           l0_W, l0_b, l1_W, l1_b, l2_W, l2_b, l3_W, l3_b, l4_W, l4_b,
           l5_W, l5_b,
           mha0_W, mha0_b, mha1_W, mha1_b, mha2_W, mha2_b, mha3_W, mha3_b,
           v1_W, v1_b, v2_W, v2_b, gnn_step, max_step, remain_step):
    f32 = jnp.float32
    src = src.astype(jnp.int32)
    w2 = w[:, :, 0]
    et0 = e_type[:, :, 0]
    et1 = e_type[:, :, 1]
    d2 = d[:, :, 0]
    pe_row = jnp.asarray(_PE)[remain_step + 0 * max_step][None, :]
    gs = jnp.asarray(gnn_step, jnp.int32).reshape(1)

    row = lambda b: jnp.asarray(b, f32).reshape(1, -1)

    vmem = pl.BlockSpec(memory_space=pltpu.VMEM)
    n_in = 28
    h, hfull, A, B, kk, vvl = pl.pallas_call(
        _prep_kernel,
        out_shape=(
            jax.ShapeDtypeStruct((N, HID), f32),
            jax.ShapeDtypeStruct((N, HD2), f32),
            jax.ShapeDtypeStruct((N, D_MODEL), f32),
            jax.ShapeDtypeStruct((N, D_MODEL), f32),
            jax.ShapeDtypeStruct((N, D_MODEL), f32),
            jax.ShapeDtypeStruct((N, D_MODEL), f32),
        ),
        in_specs=[vmem] * (n_in - 1) + [pl.BlockSpec(memory_space=pltpu.SMEM)],
        out_specs=(vmem,) * 6,
    )(x, label, h0, src, w2, et0, et1, d2, pe_row,
      l0_W, row(l0_b), l1_W, row(l1_b), l2_W, row(l2_b), l3_W, row(l3_b),
      l4_W, row(l4_b), l5_W, row(l5_b),
      mha0_W, row(mha0_b), mha1_W, row(mha1_b), mha2_W, row(mha2_b), gs)

    grid = N // _BI
    full = lambda shape: pl.BlockSpec(shape, lambda i: (0,) * len(shape))
    M = pl.pallas_call(
        _attn_kernel,
        grid=(grid,),
        in_specs=[
            full((N, D_MODEL)),
            full((N, D_MODEL)),
            full((1, D_MODEL)),
            full((N, D_MODEL)),
            full((N, D_MODEL)),
            full((D_MODEL, D_MODEL)),
            full((1, D_MODEL)),
        ],
        out_specs=pl.BlockSpec((_BI * N, D_MODEL), lambda i: (i, 0)),
        out_shape=jax.ShapeDtypeStruct((N * N, D_MODEL), f32),
    )(A, B, row(mha0_b), kk, vvl, mha3_W, row(mha3_b))

    M3 = M.reshape(N, N, D_MODEL)
    S2, Qs = pl.pallas_call(
        _combine_kernel,
        grid=(grid,),
        in_specs=[
            pl.BlockSpec((_BI * N, D_MODEL), lambda i: (i, 0)),
            pl.BlockSpec((N, _BI, D_MODEL), lambda i: (0, i, 0)),
            full((HID // 2, D_MODEL)),
            full((1, HID // 2)),
            full((1, HID // 2)),
            full((1, 1)),
        ],
        out_specs=(
            pl.BlockSpec((_BI * N, D_MODEL), lambda i: (i, 0)),
            pl.BlockSpec((_BI * N, 1), lambda i: (i, 0)),
        ),
        out_shape=(
            jax.ShapeDtypeStruct((N * N, D_MODEL), f32),
            jax.ShapeDtypeStruct((N * N, 1), f32),
        ),
    )(M, M3, v1_W, row(v1_b), row(v2_W), jnp.asarray(v2_b, f32).reshape(1, 1))

    return (S2.reshape(N * N, 1, D_MODEL), h, hfull, Qs.reshape(N * N))
